# baseline (speedup 1.0000x reference)
import functools

import jax
import jax.numpy as jnp
from jax.experimental import pallas as pl
from jax.experimental.pallas import tpu as pltpu
from jax.experimental.pallas import tpu_sc as plsc

_NUM_INTEREST = 8192
_DIM = 32
_TOPK = 8
_B = 16384
_ROWS = 512
_LANES = 128
_GATHER_WINDOW = 128
_ROWS2 = 512


def _select_kernel(x_ref, d_ref, idx_ref, dist_ref, w_ref):
    x = x_ref[...]
    d = d_ref[...]

    xn = x / jnp.maximum(
        jnp.sqrt(jnp.sum(x * x, axis=-1, keepdims=True)), 1e-8
    )
    dn = d / jnp.maximum(
        jnp.sqrt(jnp.sum(d * d, axis=-1, keepdims=True)), 1e-8
    )

    scores = jax.lax.dot_general(
        xn,
        dn,
        (((1,), (1,)), ((), ())),
        preferred_element_type=jnp.float32,
    )

    rows, k_total = scores.shape
    n_groups = k_total // _LANES
    neg_inf = jnp.float32(-jnp.inf)

    lane = jax.lax.broadcasted_iota(jnp.int32, (rows, _LANES), 1)

    v1 = v2 = v3 = jnp.full((rows, _LANES), neg_inf)
    g1 = g2 = g3 = jnp.full((rows, _LANES), jnp.int32(n_groups))
    for g in range(n_groups):
        w = scores[:, g * _LANES:(g + 1) * _LANES]
        gi = jnp.int32(g)
        b1 = w > v1
        b2 = w > v2
        b3 = w > v3
        nv1 = jnp.where(b1, w, v1)
        ng1 = jnp.where(b1, gi, g1)
        nv2 = jnp.where(b1, v1, jnp.where(b2, w, v2))
        ng2 = jnp.where(b1, g1, jnp.where(b2, gi, g2))
        nv3 = jnp.where(b2, v2, jnp.where(b3, w, v3))
        ng3 = jnp.where(b2, g2, jnp.where(b3, gi, g3))
        v1, v2, v3 = nv1, nv2, nv3
        g1, g2, g3 = ng1, ng2, ng3

    v_cand = jnp.concatenate([v1, v2, v3], axis=1)
    c_cand = jnp.concatenate(
        [g1 * _LANES + lane, g2 * _LANES + lane, g3 * _LANES + lane], axis=1
    )

    vals = []
    idxs = []
    for _ in range(_TOPK):
        m = jnp.max(v_cand, axis=1, keepdims=True)
        i = jnp.min(
            jnp.where(v_cand == m, c_cand, k_total), axis=1, keepdims=True
        )
        vals.append(m)
        idxs.append(i)
        v_cand = jnp.where(c_cand == i, neg_inf, v_cand)

    dist = jnp.concatenate(vals, axis=1)
    idx = jnp.concatenate(idxs, axis=1)

    e = jnp.exp(dist - vals[0])
    w8 = e * (1.0 / jnp.sum(e, axis=1, keepdims=True))

    idx_ref[...] = idx
    dist_ref[...] = dist
    w_ref[...] = w8


_GATHER_PAD = 128


def _sc_gather(dict_padded, idx2d, n_idx):
    mesh = plsc.VectorSubcoreMesh(
        core_axis_name="core", subcore_axis_name="subcore"
    )

    @functools.partial(
        pl.kernel,
        out_type=jax.ShapeDtypeStruct((n_idx, _GATHER_PAD), jnp.float32),
        mesh=mesh,
    )
    def gather_kernel(d_hbm, i_hbm, o_hbm):
        def body(i_vmem, o_vmem):
            pltpu.sync_copy(d_hbm.at[i_vmem.at[0]], o_vmem)

        pltpu.emit_pipeline(
            body,
            grid=(n_idx // _GATHER_WINDOW,),
            in_specs=[
                pl.BlockSpec((1, _GATHER_WINDOW), index_map=lambda i: (0, i))
            ],
            out_specs=[
                pl.BlockSpec(
                    (_GATHER_WINDOW, _GATHER_PAD), index_map=lambda i: (i, 0)
                )
            ],
            core_axis_name="subcore",
            dimension_semantics=(pltpu.PARALLEL,),
        )(i_hbm, o_hbm)

    return gather_kernel(dict_padded, idx2d)


def _combine_kernel(emb_ref, w_ref, group_ref, emb0_ref):
    emb = emb_ref[...][:, :_DIM]
    wf = w_ref[...]
    r2 = emb.shape[0] // _TOPK
    weighted = (emb * wf).reshape(r2, _TOPK, _DIM)
    group_ref[...] = jnp.sum(weighted, axis=1)
    emb0_ref[...] = emb.reshape(r2, _TOPK, _DIM)[:, 0, :]


_NSPLIT = 8


def _select(x_chunk, dictionary):
    bc, dim = x_chunk.shape
    k_total = dictionary.shape[0]
    return pl.pallas_call(
        _select_kernel,
        grid=(bc // _ROWS,),
        in_specs=[
            pl.BlockSpec((_ROWS, dim), lambda i: (i, 0)),
            pl.BlockSpec((k_total, dim), lambda i: (0, 0)),
        ],
        out_specs=[
            pl.BlockSpec((_ROWS, _TOPK), lambda i: (i, 0)),
            pl.BlockSpec((_ROWS, _TOPK), lambda i: (i, 0)),
            pl.BlockSpec((_ROWS, _TOPK), lambda i: (i, 0)),
        ],
        out_shape=[
            jax.ShapeDtypeStruct((bc, _TOPK), jnp.int32),
            jax.ShapeDtypeStruct((bc, _TOPK), jnp.float32),
            jax.ShapeDtypeStruct((bc, _TOPK), jnp.float32),
        ],
    )(x_chunk, dictionary)


def _combine(emb_flat, w8, bc, dim):
    n_idx = bc * _TOPK
    return pl.pallas_call(
        _combine_kernel,
        grid=(bc // _ROWS2,),
        in_specs=[
            pl.BlockSpec((_TOPK * _ROWS2, _GATHER_PAD), lambda i: (i, 0)),
            pl.BlockSpec((_TOPK * _ROWS2, 1), lambda i: (i, 0)),
        ],
        out_specs=[
            pl.BlockSpec((_ROWS2, dim), lambda i: (i, 0)),
            pl.BlockSpec((_ROWS2, dim), lambda i: (i, 0)),
        ],
        out_shape=[
            jax.ShapeDtypeStruct((bc, dim), jnp.float32),
            jax.ShapeDtypeStruct((bc, dim), jnp.float32),
        ],
    )(emb_flat, w8.reshape(n_idx, 1))


@jax.jit
def kernel(inputs_flatten, dictionary):
    b, dim = inputs_flatten.shape
    dict_padded = jnp.pad(dictionary, ((0, 0), (0, _GATHER_PAD - dim)))
    bc = b // _NSPLIT

    outs = []
    for s in range(_NSPLIT):
        x_chunk = jax.lax.slice_in_dim(inputs_flatten, s * bc, (s + 1) * bc)
        idx, dist, w8 = _select(x_chunk, dictionary)
        emb_flat = _sc_gather(
            dict_padded, idx.reshape(1, bc * _TOPK), bc * _TOPK
        )
        group, emb0 = _combine(emb_flat, w8, bc, dim)
        outs.append((group, idx[:, :5], dist[:, :5], emb0))

    return tuple(
        jnp.concatenate([o[i] for o in outs], axis=0) for i in range(4)
    )


# speedup vs baseline: 1.0613x; 1.0613x over previous
import functools

import jax
import jax.numpy as jnp
from jax.experimental import pallas as pl
from jax.experimental.pallas import tpu as pltpu
from jax.experimental.pallas import tpu_sc as plsc

_NUM_INTEREST = 8192
_DIM = 32
_TOPK = 8
_B = 16384
_ROWS = 512
_LANES = 128
_GATHER_WINDOW = 128
_ROWS2 = 512


def _select_kernel(x_ref, d_ref, idx_ref, dist_ref, w_ref):
    x = x_ref[...]
    d = d_ref[...]

    xn = x / jnp.maximum(
        jnp.sqrt(jnp.sum(x * x, axis=-1, keepdims=True)), 1e-8
    )
    dn = d / jnp.maximum(
        jnp.sqrt(jnp.sum(d * d, axis=-1, keepdims=True)), 1e-8
    )

    scores = jax.lax.dot_general(
        xn,
        dn,
        (((1,), (1,)), ((), ())),
        preferred_element_type=jnp.float32,
    )

    rows, k_total = scores.shape
    n_groups = k_total // _LANES
    neg_inf = jnp.float32(-jnp.inf)

    lane = jax.lax.broadcasted_iota(jnp.int32, (rows, _LANES), 1)

    v1 = v2 = v3 = jnp.full((rows, _LANES), neg_inf)
    g1 = g2 = g3 = jnp.full((rows, _LANES), jnp.int32(n_groups))
    for g in range(n_groups):
        w = scores[:, g * _LANES:(g + 1) * _LANES]
        gi = jnp.int32(g)
        b1 = w > v1
        b2 = w > v2
        b3 = w > v3
        nv1 = jnp.where(b1, w, v1)
        ng1 = jnp.where(b1, gi, g1)
        nv2 = jnp.where(b1, v1, jnp.where(b2, w, v2))
        ng2 = jnp.where(b1, g1, jnp.where(b2, gi, g2))
        nv3 = jnp.where(b2, v2, jnp.where(b3, w, v3))
        ng3 = jnp.where(b2, g2, jnp.where(b3, gi, g3))
        v1, v2, v3 = nv1, nv2, nv3
        g1, g2, g3 = ng1, ng2, ng3

    v_cand = jnp.concatenate([v1, v2, v3], axis=1)
    c_cand = jnp.concatenate(
        [g1 * _LANES + lane, g2 * _LANES + lane, g3 * _LANES + lane], axis=1
    )

    vals = []
    idxs = []
    for _ in range(_TOPK):
        m = jnp.max(v_cand, axis=1, keepdims=True)
        i = jnp.min(
            jnp.where(v_cand == m, c_cand, k_total), axis=1, keepdims=True
        )
        vals.append(m)
        idxs.append(i)
        v_cand = jnp.where(c_cand == i, neg_inf, v_cand)

    dist = jnp.concatenate(vals, axis=1)
    idx = jnp.concatenate(idxs, axis=1)

    e = jnp.exp(dist - vals[0])
    w8 = e * (1.0 / jnp.sum(e, axis=1, keepdims=True))

    idx_ref[...] = idx
    dist_ref[...] = dist
    w_ref[...] = w8


_GATHER_PAD = 128


def _sc_gather(dict_padded, idx2d, n_idx):
    mesh = plsc.VectorSubcoreMesh(
        core_axis_name="core", subcore_axis_name="subcore"
    )

    @functools.partial(
        pl.kernel,
        out_type=jax.ShapeDtypeStruct((n_idx, _GATHER_PAD), jnp.float32),
        mesh=mesh,
    )
    def gather_kernel(d_hbm, i_hbm, o_hbm):
        def body(i_vmem, o_vmem):
            pltpu.sync_copy(d_hbm.at[i_vmem.at[0]], o_vmem)

        pltpu.emit_pipeline(
            body,
            grid=(n_idx // _GATHER_WINDOW,),
            in_specs=[
                pl.BlockSpec((1, _GATHER_WINDOW), index_map=lambda i: (0, i))
            ],
            out_specs=[
                pl.BlockSpec(
                    (_GATHER_WINDOW, _GATHER_PAD), index_map=lambda i: (i, 0)
                )
            ],
            core_axis_name="subcore",
            dimension_semantics=(pltpu.PARALLEL,),
        )(i_hbm, o_hbm)

    return gather_kernel(dict_padded, idx2d)


def _combine_kernel(emb_ref, w_ref, group_ref, emb0_ref):
    emb = emb_ref[...][:, :_DIM]
    wf = w_ref[...]
    r2 = emb.shape[0] // _TOPK
    weighted = (emb * wf).reshape(r2, _TOPK, _DIM)
    group_ref[...] = jnp.sum(weighted, axis=1)
    emb0_ref[...] = emb.reshape(r2, _TOPK, _DIM)[:, 0, :]


_NSPLIT = 2


def _select(x_chunk, dictionary):
    bc, dim = x_chunk.shape
    k_total = dictionary.shape[0]
    return pl.pallas_call(
        _select_kernel,
        grid=(bc // _ROWS,),
        in_specs=[
            pl.BlockSpec((_ROWS, dim), lambda i: (i, 0)),
            pl.BlockSpec((k_total, dim), lambda i: (0, 0)),
        ],
        out_specs=[
            pl.BlockSpec((_ROWS, _TOPK), lambda i: (i, 0)),
            pl.BlockSpec((_ROWS, _TOPK), lambda i: (i, 0)),
            pl.BlockSpec((_ROWS, _TOPK), lambda i: (i, 0)),
        ],
        out_shape=[
            jax.ShapeDtypeStruct((bc, _TOPK), jnp.int32),
            jax.ShapeDtypeStruct((bc, _TOPK), jnp.float32),
            jax.ShapeDtypeStruct((bc, _TOPK), jnp.float32),
        ],
    )(x_chunk, dictionary)


def _combine(emb_flat, w8, bc, dim):
    n_idx = bc * _TOPK
    return pl.pallas_call(
        _combine_kernel,
        grid=(bc // _ROWS2,),
        in_specs=[
            pl.BlockSpec((_TOPK * _ROWS2, _GATHER_PAD), lambda i: (i, 0)),
            pl.BlockSpec((_TOPK * _ROWS2, 1), lambda i: (i, 0)),
        ],
        out_specs=[
            pl.BlockSpec((_ROWS2, dim), lambda i: (i, 0)),
            pl.BlockSpec((_ROWS2, dim), lambda i: (i, 0)),
        ],
        out_shape=[
            jax.ShapeDtypeStruct((bc, dim), jnp.float32),
            jax.ShapeDtypeStruct((bc, dim), jnp.float32),
        ],
    )(emb_flat, w8.reshape(n_idx, 1))


@jax.jit
def kernel(inputs_flatten, dictionary):
    b, dim = inputs_flatten.shape
    dict_padded = jnp.pad(dictionary, ((0, 0), (0, _GATHER_PAD - dim)))
    bc = b // _NSPLIT

    outs = []
    for s in range(_NSPLIT):
        x_chunk = jax.lax.slice_in_dim(inputs_flatten, s * bc, (s + 1) * bc)
        idx, dist, w8 = _select(x_chunk, dictionary)
        emb_flat = _sc_gather(
            dict_padded, idx.reshape(1, bc * _TOPK), bc * _TOPK
        )
        group, emb0 = _combine(emb_flat, w8, bc, dim)
        outs.append((group, idx[:, :5], dist[:, :5], emb0))

    return tuple(
        jnp.concatenate([o[i] for o in outs], axis=0) for i in range(4)
    )


# speedup vs baseline: 1.0702x; 1.0084x over previous
import functools

import jax
import jax.numpy as jnp
from jax.experimental import pallas as pl
from jax.experimental.pallas import tpu as pltpu
from jax.experimental.pallas import tpu_sc as plsc

_NUM_INTEREST = 8192
_DIM = 32
_TOPK = 8
_B = 16384
_ROWS = 512
_LANES = 128
_GATHER_WINDOW = 128
_ROWS2 = 1024


def _select_kernel(x_ref, d_ref, idx_ref, dist_ref, w_ref):
    x = x_ref[...]
    d = d_ref[...]

    xn = x / jnp.maximum(
        jnp.sqrt(jnp.sum(x * x, axis=-1, keepdims=True)), 1e-8
    )
    dn = d / jnp.maximum(
        jnp.sqrt(jnp.sum(d * d, axis=-1, keepdims=True)), 1e-8
    )

    scores = jax.lax.dot_general(
        xn,
        dn,
        (((1,), (1,)), ((), ())),
        preferred_element_type=jnp.float32,
    )

    rows, k_total = scores.shape
    n_groups = k_total // _LANES
    neg_inf = jnp.float32(-jnp.inf)

    lane = jax.lax.broadcasted_iota(jnp.int32, (rows, _LANES), 1)

    v1 = v2 = v3 = jnp.full((rows, _LANES), neg_inf)
    g1 = g2 = g3 = jnp.full((rows, _LANES), jnp.int32(n_groups))
    for g in range(n_groups):
        w = scores[:, g * _LANES:(g + 1) * _LANES]
        gi = jnp.int32(g)
        b1 = w > v1
        b2 = w > v2
        b3 = w > v3
        nv1 = jnp.where(b1, w, v1)
        ng1 = jnp.where(b1, gi, g1)
        nv2 = jnp.where(b1, v1, jnp.where(b2, w, v2))
        ng2 = jnp.where(b1, g1, jnp.where(b2, gi, g2))
        nv3 = jnp.where(b2, v2, jnp.where(b3, w, v3))
        ng3 = jnp.where(b2, g2, jnp.where(b3, gi, g3))
        v1, v2, v3 = nv1, nv2, nv3
        g1, g2, g3 = ng1, ng2, ng3

    v_cand = jnp.concatenate([v1, v2, v3], axis=1)
    c_cand = jnp.concatenate(
        [g1 * _LANES + lane, g2 * _LANES + lane, g3 * _LANES + lane], axis=1
    )

    vals = []
    idxs = []
    for _ in range(_TOPK):
        m = jnp.max(v_cand, axis=1, keepdims=True)
        i = jnp.min(
            jnp.where(v_cand == m, c_cand, k_total), axis=1, keepdims=True
        )
        vals.append(m)
        idxs.append(i)
        v_cand = jnp.where(c_cand == i, neg_inf, v_cand)

    dist = jnp.concatenate(vals, axis=1)
    idx = jnp.concatenate(idxs, axis=1)

    e = jnp.exp(dist - vals[0])
    w8 = e * (1.0 / jnp.sum(e, axis=1, keepdims=True))

    idx_ref[...] = idx
    dist_ref[...] = dist
    w_ref[...] = w8


_GATHER_PAD = 128


def _sc_gather(dict_padded, idx2d, n_idx):
    mesh = plsc.VectorSubcoreMesh(
        core_axis_name="core", subcore_axis_name="subcore"
    )

    @functools.partial(
        pl.kernel,
        out_type=jax.ShapeDtypeStruct((n_idx, _GATHER_PAD), jnp.float32),
        mesh=mesh,
    )
    def gather_kernel(d_hbm, i_hbm, o_hbm):
        def body(i_vmem, o_vmem):
            pltpu.sync_copy(d_hbm.at[i_vmem.at[0]], o_vmem)

        pltpu.emit_pipeline(
            body,
            grid=(n_idx // _GATHER_WINDOW,),
            in_specs=[
                pl.BlockSpec((1, _GATHER_WINDOW), index_map=lambda i: (0, i))
            ],
            out_specs=[
                pl.BlockSpec(
                    (_GATHER_WINDOW, _GATHER_PAD), index_map=lambda i: (i, 0)
                )
            ],
            core_axis_name="subcore",
            dimension_semantics=(pltpu.PARALLEL,),
        )(i_hbm, o_hbm)

    return gather_kernel(dict_padded, idx2d)


def _combine_kernel(emb_ref, w_ref, group_ref, emb0_ref):
    emb = emb_ref[...][:, :_DIM]
    wf = w_ref[...]
    r2 = emb.shape[0] // _TOPK
    weighted = (emb * wf).reshape(r2, _TOPK, _DIM)
    group_ref[...] = jnp.sum(weighted, axis=1)
    emb0_ref[...] = emb.reshape(r2, _TOPK, _DIM)[:, 0, :]


_NSPLIT = 4


def _select(x_chunk, dictionary):
    bc, dim = x_chunk.shape
    k_total = dictionary.shape[0]
    return pl.pallas_call(
        _select_kernel,
        grid=(bc // _ROWS,),
        in_specs=[
            pl.BlockSpec((_ROWS, dim), lambda i: (i, 0)),
            pl.BlockSpec((k_total, dim), lambda i: (0, 0)),
        ],
        out_specs=[
            pl.BlockSpec((_ROWS, _TOPK), lambda i: (i, 0)),
            pl.BlockSpec((_ROWS, _TOPK), lambda i: (i, 0)),
            pl.BlockSpec((_ROWS, _TOPK), lambda i: (i, 0)),
        ],
        out_shape=[
            jax.ShapeDtypeStruct((bc, _TOPK), jnp.int32),
            jax.ShapeDtypeStruct((bc, _TOPK), jnp.float32),
            jax.ShapeDtypeStruct((bc, _TOPK), jnp.float32),
        ],
    )(x_chunk, dictionary)


def _combine(emb_flat, w8, bc, dim):
    n_idx = bc * _TOPK
    return pl.pallas_call(
        _combine_kernel,
        grid=(bc // _ROWS2,),
        in_specs=[
            pl.BlockSpec((_TOPK * _ROWS2, _GATHER_PAD), lambda i: (i, 0)),
            pl.BlockSpec((_TOPK * _ROWS2, 1), lambda i: (i, 0)),
        ],
        out_specs=[
            pl.BlockSpec((_ROWS2, dim), lambda i: (i, 0)),
            pl.BlockSpec((_ROWS2, dim), lambda i: (i, 0)),
        ],
        out_shape=[
            jax.ShapeDtypeStruct((bc, dim), jnp.float32),
            jax.ShapeDtypeStruct((bc, dim), jnp.float32),
        ],
    )(emb_flat, w8.reshape(n_idx, 1))


@jax.jit
def kernel(inputs_flatten, dictionary):
    b, dim = inputs_flatten.shape
    dict_padded = jnp.pad(dictionary, ((0, 0), (0, _GATHER_PAD - dim)))
    bc = b // _NSPLIT

    outs = []
    for s in range(_NSPLIT):
        x_chunk = jax.lax.slice_in_dim(inputs_flatten, s * bc, (s + 1) * bc)
        idx, dist, w8 = _select(x_chunk, dictionary)
        emb_flat = _sc_gather(
            dict_padded, idx.reshape(1, bc * _TOPK), bc * _TOPK
        )
        group, emb0 = _combine(emb_flat, w8, bc, dim)
        outs.append((group, idx[:, :5], dist[:, :5], emb0))

    return tuple(
        jnp.concatenate([o[i] for o in outs], axis=0) for i in range(4)
    )


# speedup vs baseline: 1.1298x; 1.0556x over previous
import functools

import jax
import jax.numpy as jnp
from jax.experimental import pallas as pl
from jax.experimental.pallas import tpu as pltpu
from jax.experimental.pallas import tpu_sc as plsc

_NUM_INTEREST = 8192
_DIM = 32
_TOPK = 8
_B = 16384
_ROWS = 512
_LANES = 128
_GATHER_WINDOW = 128
_ROWS2 = 1024


def _select_kernel(x_ref, d_ref, idx_ref, dist_ref, w_ref, dn_ref):
    x = x_ref[...]

    @pl.when(pl.program_id(0) == 0)
    def _():
        d = d_ref[...]
        dn_ref[...] = d / jnp.maximum(
            jnp.sqrt(jnp.sum(d * d, axis=-1, keepdims=True)), 1e-8
        )

    xn = x / jnp.maximum(
        jnp.sqrt(jnp.sum(x * x, axis=-1, keepdims=True)), 1e-8
    )
    dn = dn_ref[...]

    scores = jax.lax.dot_general(
        xn,
        dn,
        (((1,), (1,)), ((), ())),
        preferred_element_type=jnp.float32,
    )

    rows, k_total = scores.shape
    n_groups = k_total // _LANES
    neg_inf = jnp.float32(-jnp.inf)

    lane = jax.lax.broadcasted_iota(jnp.int32, (rows, _LANES), 1)

    v1 = v2 = v3 = jnp.full((rows, _LANES), neg_inf)
    g1 = g2 = g3 = jnp.full((rows, _LANES), jnp.int32(n_groups))
    for g in range(n_groups):
        w = scores[:, g * _LANES:(g + 1) * _LANES]
        gi = jnp.int32(g)
        b1 = w > v1
        b2 = w > v2
        b3 = w > v3
        nv1 = jnp.where(b1, w, v1)
        ng1 = jnp.where(b1, gi, g1)
        nv2 = jnp.where(b1, v1, jnp.where(b2, w, v2))
        ng2 = jnp.where(b1, g1, jnp.where(b2, gi, g2))
        nv3 = jnp.where(b2, v2, jnp.where(b3, w, v3))
        ng3 = jnp.where(b2, g2, jnp.where(b3, gi, g3))
        v1, v2, v3 = nv1, nv2, nv3
        g1, g2, g3 = ng1, ng2, ng3

    v_cand = jnp.concatenate([v1, v2, v3], axis=1)
    c_cand = jnp.concatenate(
        [g1 * _LANES + lane, g2 * _LANES + lane, g3 * _LANES + lane], axis=1
    )

    vals = []
    idxs = []
    for _ in range(_TOPK):
        m = jnp.max(v_cand, axis=1, keepdims=True)
        i = jnp.min(
            jnp.where(v_cand == m, c_cand, k_total), axis=1, keepdims=True
        )
        vals.append(m)
        idxs.append(i)
        v_cand = jnp.where(c_cand == i, neg_inf, v_cand)

    dist = jnp.concatenate(vals, axis=1)
    idx = jnp.concatenate(idxs, axis=1)

    e = jnp.exp(dist - vals[0])
    w8 = e * (1.0 / jnp.sum(e, axis=1, keepdims=True))

    idx_ref[...] = idx
    dist_ref[...] = dist
    w_ref[...] = w8


_GATHER_PAD = 128


def _sc_gather(dict_padded, idx2d, n_idx):
    mesh = plsc.VectorSubcoreMesh(
        core_axis_name="core", subcore_axis_name="subcore"
    )

    @functools.partial(
        pl.kernel,
        out_type=jax.ShapeDtypeStruct((n_idx, _GATHER_PAD), jnp.float32),
        mesh=mesh,
    )
    def gather_kernel(d_hbm, i_hbm, o_hbm):
        def body(i_vmem, o_vmem):
            pltpu.sync_copy(d_hbm.at[i_vmem.at[0]], o_vmem)

        pltpu.emit_pipeline(
            body,
            grid=(n_idx // _GATHER_WINDOW,),
            in_specs=[
                pl.BlockSpec((1, _GATHER_WINDOW), index_map=lambda i: (0, i))
            ],
            out_specs=[
                pl.BlockSpec(
                    (_GATHER_WINDOW, _GATHER_PAD), index_map=lambda i: (i, 0)
                )
            ],
            core_axis_name="subcore",
            dimension_semantics=(pltpu.PARALLEL,),
        )(i_hbm, o_hbm)

    return gather_kernel(dict_padded, idx2d)


def _combine_kernel(emb_ref, w_ref, group_ref, emb0_ref):
    emb = emb_ref[...][:, :_DIM]
    wf = w_ref[...]
    r2 = emb.shape[0] // _TOPK
    weighted = (emb * wf).reshape(r2, _TOPK, _DIM)
    group_ref[...] = jnp.sum(weighted, axis=1)
    emb0_ref[...] = emb.reshape(r2, _TOPK, _DIM)[:, 0, :]


_NSPLIT = 4


def _select(x_chunk, dictionary):
    bc, dim = x_chunk.shape
    k_total = dictionary.shape[0]
    return pl.pallas_call(
        _select_kernel,
        grid=(bc // _ROWS,),
        in_specs=[
            pl.BlockSpec((_ROWS, dim), lambda i: (i, 0)),
            pl.BlockSpec((k_total, dim), lambda i: (0, 0)),
        ],
        out_specs=[
            pl.BlockSpec((_ROWS, _TOPK), lambda i: (i, 0)),
            pl.BlockSpec((_ROWS, _TOPK), lambda i: (i, 0)),
            pl.BlockSpec((_ROWS, _TOPK), lambda i: (i, 0)),
        ],
        out_shape=[
            jax.ShapeDtypeStruct((bc, _TOPK), jnp.int32),
            jax.ShapeDtypeStruct((bc, _TOPK), jnp.float32),
            jax.ShapeDtypeStruct((bc, _TOPK), jnp.float32),
        ],
        scratch_shapes=[pltpu.VMEM((k_total, dim), jnp.float32)],
    )(x_chunk, dictionary)


def _combine(emb_flat, w8, bc, dim):
    n_idx = bc * _TOPK
    return pl.pallas_call(
        _combine_kernel,
        grid=(bc // _ROWS2,),
        in_specs=[
            pl.BlockSpec((_TOPK * _ROWS2, _GATHER_PAD), lambda i: (i, 0)),
            pl.BlockSpec((_TOPK * _ROWS2, 1), lambda i: (i, 0)),
        ],
        out_specs=[
            pl.BlockSpec((_ROWS2, dim), lambda i: (i, 0)),
            pl.BlockSpec((_ROWS2, dim), lambda i: (i, 0)),
        ],
        out_shape=[
            jax.ShapeDtypeStruct((bc, dim), jnp.float32),
            jax.ShapeDtypeStruct((bc, dim), jnp.float32),
        ],
    )(emb_flat, w8.reshape(n_idx, 1))


@jax.jit
def kernel(inputs_flatten, dictionary):
    b, dim = inputs_flatten.shape
    dict_padded = jnp.pad(dictionary, ((0, 0), (0, _GATHER_PAD - dim)))
    bc = b // _NSPLIT

    outs = []
    for s in range(_NSPLIT):
        x_chunk = jax.lax.slice_in_dim(inputs_flatten, s * bc, (s + 1) * bc)
        idx, dist, w8 = _select(x_chunk, dictionary)
        emb_flat = _sc_gather(
            dict_padded, idx.reshape(1, bc * _TOPK), bc * _TOPK
        )
        group, emb0 = _combine(emb_flat, w8, bc, dim)
        outs.append((group, idx[:, :5], dist[:, :5], emb0))

    return tuple(
        jnp.concatenate([o[i] for o in outs], axis=0) for i in range(4)
    )
